# baseline (speedup 1.0000x reference)
import functools
import math

import jax
import jax.numpy as jnp
from jax import lax
from jax.experimental import pallas as pl
from jax.experimental.pallas import tpu as pltpu
from jax.experimental.pallas import tpu_sc as plsc

VOCAB = 1_000_000
D = 64
SCALE = math.sqrt(D)

_info = plsc.get_sparse_core_info()
NC = _info.num_cores
NS = _info.num_subcores
NW = NC * NS
L = _info.num_lanes

CHUNK = 128
NBUF = 2


def _build(B):
    per_w = B // NW
    nch = per_w // CHUNK
    nouter = nch // NBUF

    mesh = plsc.VectorSubcoreMesh(core_axis_name="c", subcore_axis_name="s")

    @functools.partial(
        pl.kernel,
        mesh=mesh,
        compiler_params=pltpu.CompilerParams(use_tc_tiling_on_sc=False),
        out_type=jax.ShapeDtypeStruct((B, D), jnp.float32),
        scratch_types=[
            pltpu.VMEM((nch, CHUNK), jnp.int32),
            [pltpu.VMEM((CHUNK, D), jnp.float32) for _ in range(NBUF)],
            [pltpu.VMEM((CHUNK, D), jnp.float32) for _ in range(NBUF)],
            [pltpu.SemaphoreType.DMA for _ in range(NBUF)],
            [pltpu.SemaphoreType.DMA for _ in range(NBUF)],
        ],
    )
    def emb(tok_hbm, table_hbm, out_hbm, idx_v, bin_, bout, gsem, osem):
        wid = lax.axis_index("s") * NC + lax.axis_index("c")
        pltpu.sync_copy(tok_hbm.at[pl.ds(wid * nch, nch)], idx_v)
        row0 = wid * per_w

        def gather(j, b):
            pltpu.async_copy(table_hbm.at[idx_v.at[j]], bin_[b], gsem[b])

        def put(j, b):
            pltpu.async_copy(bout[b], out_hbm.at[pl.ds(row0 + j * CHUNK, CHUNK)],
                             osem[b])

        for b in range(NBUF):
            gather(b, b)

        def outer(jj, _):
            for b in range(NBUF):
                j = jj * NBUF + b
                pltpu.make_async_copy(table_hbm.at[idx_v.at[j]], bin_[b],
                                      gsem[b]).wait()

                @pl.when(jj > 0)
                def _():
                    pltpu.make_async_copy(
                        bout[b], out_hbm.at[pl.ds(row0, CHUNK)], osem[b]).wait()

                def scale_body(r4, _):
                    for rr in range(4):
                        r = r4 * 4 + rr
                        for v in range(D // L):
                            sl = pl.ds(v * L, L)
                            bout[b][r, sl] = bin_[b][r, sl] * SCALE
                    return ()

                lax.fori_loop(0, CHUNK // 4, scale_body, (), unroll=4)
                put(j, b)

                @pl.when(jj < nouter - 1)
                def _():
                    gather(j + NBUF, b)
            return ()

        lax.fori_loop(0, nouter, outer, ())
        for b in range(NBUF):
            pltpu.make_async_copy(bout[b], out_hbm.at[pl.ds(row0, CHUNK)],
                                  osem[b]).wait()

    return emb


def kernel(tokens, table):
    S, T = tokens.shape
    B = S * T
    tok2d = tokens.reshape(B // CHUNK, CHUNK).astype(jnp.int32)
    out = _build(B)(tok2d, table)
    return out.reshape(S, T, D)
